# Initial kernel scaffold; baseline (speedup 1.0000x reference)
import jax
import jax.numpy as jnp
from jax.experimental import pallas as pl


def kernel(real_images, fake_images, fake_outputs):
    raise NotImplementedError("write your pallas kernel here")


# speedup vs baseline: 16.8920x; 16.8920x over previous
import functools

import jax
import jax.numpy as jnp
from jax import lax
from jax.experimental import pallas as pl
from jax.experimental.pallas import tpu as pltpu

_N = 1024
_INF = float("inf")


def _three_smallest(d2):
    m1 = jnp.min(d2, axis=1)
    eq1 = d2 == m1[:, None]
    c1 = jnp.sum(eq1.astype(jnp.float32), axis=1)
    d2b = jnp.where(eq1, _INF, d2)
    m2r = jnp.min(d2b, axis=1)
    eq2 = d2b == m2r[:, None]
    c2 = jnp.sum(eq2.astype(jnp.float32), axis=1)
    second = jnp.where(c1 >= 2.0, m1, m2r)
    d2c = jnp.where(eq2, _INF, d2b)
    m3r = jnp.min(d2c, axis=1)
    third = jnp.where(
        c1 >= 3.0, m1,
        jnp.where(c1 == 2.0, m2r, jnp.where(c2 >= 2.0, m2r, m3r)))
    return m1, second, third


def _d2mat(x, y):
    xr = x.reshape(_N, 1)
    yr = y.reshape(_N, 1)
    dx = xr - x.reshape(1, _N)
    dy = yr - y.reshape(1, _N)
    return dx * dx + dy * dy


def _body(fx_ref, fy_ref, fr_ref, rx_ref, ry_ref,
          fm1_ref, fm2_ref, fm3_ref, rm1_ref, rm2_ref, rm3_ref,
          feas_ref, sumr_ref):
    fx = fx_ref[0, 0, :]
    fy = fy_ref[0, 0, :]
    d2f = _d2mat(fx, fy)

    a, b, c = _three_smallest(d2f)
    fm1_ref[0, 0, :] = jnp.sqrt(a)
    fm2_ref[0, 0, :] = jnp.sqrt(b)
    fm3_ref[0, 0, :] = jnp.sqrt(c)

    dist = jnp.sqrt(d2f)
    r = jnp.abs(fr_ref[0, 0, :])
    radii = r.reshape(_N, 1) + r.reshape(1, _N)
    row_i = lax.broadcasted_iota(jnp.int32, (_N, _N), 0)
    col_j = lax.broadcasted_iota(jnp.int32, (_N, _N), 1)
    valid = (col_j < row_i) & (dist > 0.0)
    ov = jnp.maximum(radii - (dist + 0.0001), 0.0)
    feas_ref[0, 0, :] = jnp.full((128,), jnp.sum(jnp.where(valid, ov, 0.0)))
    sumr_ref[0, 0, :] = jnp.full((128,), jnp.sum(r))

    d2r = _d2mat(rx_ref[0, 0, :], ry_ref[0, 0, :])
    a, b, c = _three_smallest(d2r)
    rm1_ref[0, 0, :] = jnp.sqrt(a)
    rm2_ref[0, 0, :] = jnp.sqrt(b)
    rm3_ref[0, 0, :] = jnp.sqrt(c)


def _main_call(fx, fy, fr, rx, ry, interpret=False):
    B = fx.shape[0]
    row = pl.BlockSpec((1, 1, _N), lambda b: (b, 0, 0))
    lane = pl.BlockSpec((1, 1, 128), lambda b: (b, 0, 0))
    outs = [jax.ShapeDtypeStruct((B, 1, _N), jnp.float32)] * 6 + \
           [jax.ShapeDtypeStruct((B, 1, 128), jnp.float32)] * 2
    ins = [a.reshape(B, 1, _N) for a in (fx, fy, fr, rx, ry)]
    res = pl.pallas_call(
        _body,
        grid=(B,),
        in_specs=[row] * 5,
        out_specs=[row] * 6 + [lane] * 2,
        out_shape=outs,
        compiler_params=pltpu.CompilerParams(
            dimension_semantics=("arbitrary",)),
        interpret=interpret,
    )(*ins)
    return [a.reshape(B, -1) for a in res]


def _quantile_sorted(s, q):
    n = s.shape[-1]
    pos = q * (n - 1)
    lo = jnp.floor(pos).astype(jnp.int32)
    hi = jnp.ceil(pos).astype(jnp.int32)
    frac = pos - lo.astype(jnp.float32)
    vlo = jnp.take(s, lo, axis=-1)
    vhi = jnp.take(s, hi, axis=-1)
    if s.ndim == 2:
        frac = frac[None, :]
    return vlo + (vhi - vlo) * frac


def _mse(a, b):
    return jnp.mean((a - b) ** 2)


def kernel(real_images, fake_images, fake_outputs, interpret=False):
    fx = fake_images[:, :, 0]
    fy = fake_images[:, :, 1]
    fr = fake_images[:, :, 2]
    rx = real_images[:, :, 0]
    ry = real_images[:, :, 1]
    rr = real_images[:, :, 2]

    fm1, fm2, fm3, rm1, rm2, rm3, feas, sumr = _main_call(
        fx, fy, fr, rx, ry, interpret=interpret)

    feas_loss = jnp.sum(feas[:, 0]) / (jnp.sum(sumr[:, 0]) * jnp.float32(_N))

    q7 = jnp.array([0.05, 0.1, 0.25, 0.5, 0.75, 0.9, 0.95], dtype=jnp.float32)
    q5 = jnp.array([0.05, 0.25, 0.5, 0.75, 0.95], dtype=jnp.float32)
    q3 = jnp.array([0.05, 0.5, 0.95], dtype=jnp.float32)

    radius_loss = _mse(_quantile_sorted(jnp.sort(fr.ravel()), q7),
                       _quantile_sorted(jnp.sort(rr.ravel()), q7))
    grid_loss = (_mse(_quantile_sorted(jnp.sort(fx.ravel()), q5),
                      _quantile_sorted(jnp.sort(rx.ravel()), q5))
                 + _mse(_quantile_sorted(jnp.sort(fy.ravel()), q5),
                        _quantile_sorted(jnp.sort(ry.ravel()), q5))) / 2.0

    knn_f = jnp.sort(jnp.concatenate([fm1, fm2, fm3], axis=1), axis=1)
    knn_r = jnp.sort(jnp.concatenate([rm1, rm2, rm3], axis=1), axis=1)
    qf = _quantile_sorted(knn_f, q3).T
    qr = _quantile_sorted(knn_r, q3).T
    distance_loss = _mse(qf, qr)

    labels = jnp.ones_like(fake_outputs) - 0.1
    logp = jnp.maximum(jnp.log(fake_outputs), -100.0)
    log1mp = jnp.maximum(jnp.log(1.0 - fake_outputs), -100.0)
    gan_loss = -jnp.mean(labels * logp + (1.0 - labels) * log1mp)

    return radius_loss + feas_loss + gan_loss + grid_loss + distance_loss


# speedup vs baseline: 18.4952x; 1.0949x over previous
import functools

import numpy as np
import jax
import jax.numpy as jnp
from jax import lax
from jax.experimental import pallas as pl
from jax.experimental.pallas import tpu as pltpu

_N = 1024
_INF = float("inf")


def _two_next_smallest(d2):
    eq1 = d2 == 0.0
    c1 = jnp.sum(eq1.astype(jnp.float32), axis=0)
    d2b = jnp.where(eq1, _INF, d2)
    m2r = jnp.min(d2b, axis=0)
    eq2 = d2b == m2r[None, :]
    c2 = jnp.sum(eq2.astype(jnp.float32), axis=0)
    second = jnp.where(c1 >= 2.0, 0.0, m2r)
    d2c = jnp.where(eq2, _INF, d2b)
    m3r = jnp.min(d2c, axis=0)
    third = jnp.where(
        c1 >= 3.0, 0.0,
        jnp.where(c1 == 2.0, m2r, jnp.where(c2 >= 2.0, m2r, m3r)))
    return second, third


def _d2mat(x, y):
    dx = x.reshape(_N, 1) - x.reshape(1, _N)
    dy = y.reshape(_N, 1) - y.reshape(1, _N)
    return dx * dx + dy * dy


def _body(fx_ref, fy_ref, fr_ref, rx_ref, ry_ref,
          fm2_ref, fm3_ref, rm2_ref, rm3_ref, feas_ref, sumr_ref):
    fx = fx_ref[0, 0, :]
    fy = fy_ref[0, 0, :]
    d2f = _d2mat(fx, fy)

    b, c = _two_next_smallest(d2f)
    fm2_ref[0, 0, :] = jnp.sqrt(b)
    fm3_ref[0, 0, :] = jnp.sqrt(c)

    dist = jnp.sqrt(d2f)
    r = jnp.abs(fr_ref[0, 0, :])
    radii = r.reshape(_N, 1) + r.reshape(1, _N)
    ov = jnp.maximum(radii - (dist + 0.0001), 0.0)
    total = jnp.sum(jnp.where(d2f > 0.0, ov, 0.0))
    feas_ref[0, 0, :] = jnp.full((128,), 0.5 * total)
    sumr_ref[0, 0, :] = jnp.full((128,), jnp.sum(r))

    d2r = _d2mat(rx_ref[0, 0, :], ry_ref[0, 0, :])
    b, c = _two_next_smallest(d2r)
    rm2_ref[0, 0, :] = jnp.sqrt(b)
    rm3_ref[0, 0, :] = jnp.sqrt(c)


def _main_call(fx, fy, fr, rx, ry, interpret=False):
    B = fx.shape[0]
    row = pl.BlockSpec((1, 1, _N), lambda b: (b, 0, 0))
    lane = pl.BlockSpec((1, 1, 128), lambda b: (b, 0, 0))
    outs = [jax.ShapeDtypeStruct((B, 1, _N), jnp.float32)] * 4 + \
           [jax.ShapeDtypeStruct((B, 1, 128), jnp.float32)] * 2
    ins = [a.reshape(B, 1, _N) for a in (fx, fy, fr, rx, ry)]
    res = pl.pallas_call(
        _body,
        grid=(B,),
        in_specs=[row] * 5,
        out_specs=[row] * 4 + [lane] * 2,
        out_shape=outs,
        compiler_params=pltpu.CompilerParams(
            dimension_semantics=("arbitrary",)),
        interpret=interpret,
    )(*ins)
    return [a.reshape(B, -1) for a in res]


def _qpos(q, n):
    pos = np.float32(q) * np.float32(n - 1)
    lo = int(np.floor(pos))
    return lo, float(pos - np.float32(lo))


def _quantile_sorted(s, qs, n_virtual=None):
    n = s.shape[-1]
    shift = 0
    if n_virtual is not None:
        shift = n_virtual - n
        n = n_virtual
    vals = []
    for q in qs:
        lo, frac = _qpos(q, n)
        lo -= shift
        if lo + 1 < 0:
            vals.append(jnp.zeros(s.shape[:-1], s.dtype))
            continue
        vlo = s[..., lo] if lo >= 0 else jnp.zeros(s.shape[:-1], s.dtype)
        vhi = s[..., lo + 1]
        vals.append(vlo + (vhi - vlo) * jnp.float32(frac))
    return jnp.stack(vals, axis=0)


def _mse(a, b):
    return jnp.mean((a - b) ** 2)


def kernel(real_images, fake_images, fake_outputs, interpret=False):
    fx = fake_images[:, :, 0]
    fy = fake_images[:, :, 1]
    fr = fake_images[:, :, 2]
    rx = real_images[:, :, 0]
    ry = real_images[:, :, 1]
    rr = real_images[:, :, 2]

    fm2, fm3, rm2, rm3, feas, sumr = _main_call(
        fx, fy, fr, rx, ry, interpret=interpret)

    feas_loss = jnp.sum(feas[:, 0]) / (jnp.sum(sumr[:, 0]) * jnp.float32(_N))

    q7 = [0.05, 0.1, 0.25, 0.5, 0.75, 0.9, 0.95]
    q5 = [0.05, 0.25, 0.5, 0.75, 0.95]
    q3 = [0.05, 0.5, 0.95]

    radius_loss = _mse(_quantile_sorted(jnp.sort(fr.ravel()), q7),
                       _quantile_sorted(jnp.sort(rr.ravel()), q7))
    grid_loss = (_mse(_quantile_sorted(jnp.sort(fx.ravel()), q5),
                      _quantile_sorted(jnp.sort(rx.ravel()), q5))
                 + _mse(_quantile_sorted(jnp.sort(fy.ravel()), q5),
                        _quantile_sorted(jnp.sort(ry.ravel()), q5))) / 2.0

    knn_f = jnp.sort(jnp.concatenate([fm2, fm3], axis=1), axis=1)
    knn_r = jnp.sort(jnp.concatenate([rm2, rm3], axis=1), axis=1)
    qf = _quantile_sorted(knn_f, q3, n_virtual=3 * _N)
    qr = _quantile_sorted(knn_r, q3, n_virtual=3 * _N)
    distance_loss = _mse(qf, qr)

    labels = jnp.ones_like(fake_outputs) - 0.1
    logp = jnp.maximum(jnp.log(fake_outputs), -100.0)
    log1mp = jnp.maximum(jnp.log(1.0 - fake_outputs), -100.0)
    gan_loss = -jnp.mean(labels * logp + (1.0 - labels) * log1mp)

    return radius_loss + feas_loss + gan_loss + grid_loss + distance_loss


# speedup vs baseline: 29.8391x; 1.6133x over previous
import functools

import numpy as np
import jax
import jax.numpy as jnp
from jax import lax
from jax.experimental import pallas as pl
from jax.experimental.pallas import tpu as pltpu

_N = 1024
_INF = float("inf")
_IMIN = -(2 ** 31)
_IMAX = 2 ** 31 - 1


def _two_next_smallest(d2):
    eq1 = d2 == 0.0
    c1 = jnp.sum(eq1.astype(jnp.float32), axis=0)
    d2b = jnp.where(eq1, _INF, d2)
    m2r = jnp.min(d2b, axis=0)
    eq2 = d2b == m2r[None, :]
    c2 = jnp.sum(eq2.astype(jnp.float32), axis=0)
    second = jnp.where(c1 >= 2.0, 0.0, m2r)
    d2c = jnp.where(eq2, _INF, d2b)
    m3r = jnp.min(d2c, axis=0)
    third = jnp.where(
        c1 >= 3.0, 0.0,
        jnp.where(c1 == 2.0, m2r, jnp.where(c2 >= 2.0, m2r, m3r)))
    return second, third


def _d2mat(x, y):
    dx = x.reshape(_N, 1) - x.reshape(1, _N)
    dy = y.reshape(_N, 1) - y.reshape(1, _N)
    return dx * dx + dy * dy


def _dist_body(fx_ref, fy_ref, fr_ref, rx_ref, ry_ref,
               fm2_ref, fm3_ref, rm2_ref, rm3_ref, feas_ref, sumr_ref):
    fx = fx_ref[0, 0, :]
    fy = fy_ref[0, 0, :]
    d2f = _d2mat(fx, fy)

    b, c = _two_next_smallest(d2f)
    fm2_ref[0, 0, :] = jnp.sqrt(b)
    fm3_ref[0, 0, :] = jnp.sqrt(c)

    dist = jnp.sqrt(d2f)
    r = jnp.abs(fr_ref[0, 0, :])
    radii = r.reshape(_N, 1) + r.reshape(1, _N)
    ov = jnp.maximum(radii - (dist + 0.0001), 0.0)
    total = jnp.sum(jnp.where(d2f > 0.0, ov, 0.0))
    feas_ref[0, 0, :] = jnp.full((128,), 0.5 * total)
    sumr_ref[0, 0, :] = jnp.full((128,), jnp.sum(r))

    d2r = _d2mat(rx_ref[0, 0, :], ry_ref[0, 0, :])
    b, c = _two_next_smallest(d2r)
    rm2_ref[0, 0, :] = jnp.sqrt(b)
    rm3_ref[0, 0, :] = jnp.sqrt(c)


def _dist_call(fx, fy, fr, rx, ry, interpret=False):
    B = fx.shape[0]
    row = pl.BlockSpec((1, 1, _N), lambda b: (b, 0, 0))
    lane = pl.BlockSpec((1, 1, 128), lambda b: (b, 0, 0))
    outs = [jax.ShapeDtypeStruct((B, 1, _N), jnp.float32)] * 4 + \
           [jax.ShapeDtypeStruct((B, 1, 128), jnp.float32)] * 2
    ins = [a.reshape(B, 1, _N) for a in (fx, fy, fr, rx, ry)]
    res = pl.pallas_call(
        _dist_body,
        grid=(B,),
        in_specs=[row] * 5,
        out_specs=[row] * 4 + [lane] * 2,
        out_shape=outs,
        compiler_params=pltpu.CompilerParams(
            dimension_semantics=("arbitrary",)),
        interpret=interpret,
    )(*ins)
    return [a.reshape(B, -1) for a in res]


def _qpos(q, n):
    pos = np.float32(q) * np.float32(n - 1)
    lo = int(np.floor(pos))
    return lo, float(pos - np.float32(lo))


def _to_ukey(f):
    b = lax.bitcast_convert_type(f, jnp.int32)
    key = b ^ ((b >> 31) & jnp.int32(0x7FFFFFFF))
    return key ^ jnp.int32(_IMIN)


def _key_to_f32(key):
    b = key ^ ((key >> 31) & jnp.int32(0x7FFFFFFF))
    return lax.bitcast_convert_type(b, jnp.float32)


def _bisect(data_u, ranks, count):
    def step(pi, carry):
        p = 31 - pi
        res, rem = carry
        sp = jnp.right_shift(data_u, p)
        bit = jnp.left_shift(jnp.int32(1), p)
        nres, nrem = [], []
        for r, m in zip(res, rem):
            cnt = count(sp == jnp.right_shift(r, p))
            go1 = m >= cnt
            nres.append(jnp.where(go1, jnp.bitwise_or(r, bit), r))
            nrem.append(jnp.where(go1, m - cnt, m))
        return tuple(nres), tuple(nrem)

    res0 = tuple(jnp.zeros_like(r) for r in ranks)
    res, _ = lax.fori_loop(0, 32, step, (res0, tuple(ranks)))
    return list(res)


def _pair_from_lo(skeys, ukey_lo, lo_rank, count, reduce_min):
    klo = ukey_lo ^ jnp.int32(_IMIN)
    cnt = count(skeys <= klo)
    succ = reduce_min(jnp.where(skeys > klo, skeys, jnp.int32(_IMAX)))
    khi = jnp.where(cnt >= lo_rank + 2, klo, succ)
    return _key_to_f32(klo), _key_to_f32(khi)


def _interp(vlo, vhi, frac):
    return vlo + (vhi - vlo) * jnp.float32(frac)


_Q7 = [0.05, 0.1, 0.25, 0.5, 0.75, 0.9, 0.95]
_Q5 = [0.05, 0.25, 0.5, 0.75, 0.95]


def _channel_quantiles(data_f32, qs):
    n = _N * 16
    pos = [_qpos(q, n) for q in qs]
    data_u = _to_ukey(data_f32)
    skeys = data_u ^ jnp.int32(_IMIN)
    count = lambda pred: jnp.sum(pred.astype(jnp.int32))
    ukeys = _bisect(data_u, [jnp.int32(lo) for lo, _ in pos], count)
    out = []
    for (lo, frac), uk in zip(pos, ukeys):
        vlo, vhi = _pair_from_lo(skeys, uk, lo, count, jnp.min)
        out.append(_interp(vlo, vhi, frac))
    return out


def _knn_quantiles(knn_u):
    lo50, frac50 = _qpos(0.5, 3 * _N)
    lo95, frac95 = _qpos(0.95, 3 * _N)
    d50, d95 = lo50 - _N, lo95 - _N

    def count(pred):
        s = jnp.sum(pred.astype(jnp.int32), axis=2, keepdims=True)
        return jnp.sum(s, axis=0, keepdims=True)

    def reduce_min(x):
        s = jnp.min(x, axis=2, keepdims=True)
        return jnp.min(s, axis=0, keepdims=True)

    skeys = knn_u ^ jnp.int32(_IMIN)
    r0 = jnp.zeros((1, 32, 1), jnp.int32)
    ukeys = _bisect(knn_u, [r0 + d50, r0 + d95], count)
    v50 = _interp(*_pair_from_lo(skeys, ukeys[0], d50, count, reduce_min),
                  frac50)
    v95 = _interp(*_pair_from_lo(skeys, ukeys[1], d95, count, reduce_min),
                  frac95)
    return v50, v95


def _loss_body(ch_ref, knn_ref, feas_ref, sumr_ref, fo_ref, out_ref):
    qfr = _channel_quantiles(ch_ref[0], _Q7)
    qrr = _channel_quantiles(ch_ref[1], _Q7)
    radius_loss = sum((a - b) ** 2 for a, b in zip(qfr, qrr)) / 7.0

    qfx = _channel_quantiles(ch_ref[2], _Q5)
    qrx = _channel_quantiles(ch_ref[3], _Q5)
    qfy = _channel_quantiles(ch_ref[4], _Q5)
    qry = _channel_quantiles(ch_ref[5], _Q5)
    grid_loss = (sum((a - b) ** 2 for a, b in zip(qfx, qrx)) / 5.0
                 + sum((a - b) ** 2 for a, b in zip(qfy, qry)) / 5.0) / 2.0

    knn_u = _to_ukey(knn_ref[...])
    v50, v95 = _knn_quantiles(knn_u)
    d50 = v50[:, 0:16, :] - v50[:, 16:32, :]
    d95 = v95[:, 0:16, :] - v95[:, 16:32, :]
    distance_loss = (jnp.sum(d50 * d50) + jnp.sum(d95 * d95)) / 48.0

    feas_loss = jnp.sum(feas_ref[:, 0:1]) / (
        jnp.sum(sumr_ref[:, 0:1]) * jnp.float32(_N))

    p = fo_ref[0, :]
    logp = jnp.maximum(jnp.log(p), -100.0)
    log1mp = jnp.maximum(jnp.log(1.0 - p), -100.0)
    gan_loss = -jnp.mean(0.9 * logp + 0.1 * log1mp)

    total = radius_loss + feas_loss + gan_loss + grid_loss + distance_loss
    out_ref[:, :] = jnp.full((8, 128), total)


def _loss_call(ch, knn, feas, sumr, fo, interpret=False):
    return pl.pallas_call(
        _loss_body,
        out_shape=jax.ShapeDtypeStruct((8, 128), jnp.float32),
        interpret=interpret,
    )(ch, knn, feas, sumr, fo)


def kernel(real_images, fake_images, fake_outputs, interpret=False):
    B = real_images.shape[0]
    fx = fake_images[:, :, 0]
    fy = fake_images[:, :, 1]
    fr = fake_images[:, :, 2]
    rx = real_images[:, :, 0]
    ry = real_images[:, :, 1]
    rr = real_images[:, :, 2]

    fm2, fm3, rm2, rm3, feas, sumr = _dist_call(
        fx, fy, fr, rx, ry, interpret=interpret)

    ch = jnp.stack([fr, rr, fx, rx, fy, ry]).reshape(6, 128, 128)
    fake2048 = jnp.concatenate([fm2, fm3], axis=1).reshape(B, 16, 128)
    real2048 = jnp.concatenate([rm2, rm3], axis=1).reshape(B, 16, 128)
    knn = jnp.concatenate(
        [fake2048.transpose(1, 0, 2), real2048.transpose(1, 0, 2)], axis=1)

    out = _loss_call(ch, knn, feas, sumr,
                     fake_outputs.reshape(1, B), interpret=interpret)
    return out[0, 0]
